# baseline (speedup 1.0000x reference)
import jax
import jax.numpy as jnp
from jax import lax
from jax.experimental import pallas as pl
from jax.experimental.pallas import tpu as pltpu
from jax.experimental.pallas import tpu_sc as plsc

N = 10000
E = 320000
D = 128
DH = D // 2
NS = 16
B = 125
W = E // NS // B
CH = 40
NCHUNK = W // CH
NBUF = 4
ROWS_PER_TILE = N // NS
ZROWS = 125


def _body(x2_hbm, col_hbm, cols_hbm, row_hbm, out_hbm,
          x_sh, acc_sh, colbuf, colbuf_s, rowbuf, msg,
          gsem0, gsem1, gsem2, gsem3, ssem0, ssem1, ssem2, ssem3):
    c = lax.axis_index("c")
    s = lax.axis_index("s")
    r0 = s * ROWS_PER_TILE
    c0 = c * DH

    pltpu.sync_copy(x2_hbm.at[pl.ds(c * N + r0, ROWS_PER_TILE)],
                    x_sh.at[pl.ds(r0, ROWS_PER_TILE)])

    zeros16 = jnp.zeros((16,), jnp.float32)

    def _zero_row(r, carry):
        for k in range(DH // 16):
            msg[0, r, pl.ds(k * 16, 16)] = zeros16
        return carry

    lax.fori_loop(0, ZROWS, _zero_row, 0)
    for b in range(ROWS_PER_TILE // ZROWS):
        pltpu.sync_copy(msg.at[0], acc_sh.at[pl.ds(r0 + b * ZROWS, ZROWS)])

    plsc.subcore_barrier()

    gsems = (gsem0, gsem1, gsem2, gsem3)
    ssems = (ssem0, ssem1, ssem2, ssem3)

    def _start_gather(w, q):
        pltpu.async_copy(x2_hbm.at[colbuf.at[w]], msg.at[q], gsems[q])

    def _wait_gather(w, q):
        pltpu.make_async_copy(x2_hbm.at[colbuf.at[w]], msg.at[q],
                              gsems[q]).wait()

    def _start_gather_s(j, q):
        pltpu.async_copy(x_sh.at[colbuf_s.at[j]], msg.at[q], gsems[q])

    def _wait_gather_s(j, q):
        pltpu.make_async_copy(x_sh.at[colbuf_s.at[j]], msg.at[q],
                              gsems[q]).wait()

    def _start_scatter(w, q):
        pltpu.async_copy(msg.at[q], acc_sh.at[rowbuf.at[w]], ssems[q],
                         add=True)

    def _wait_scatter(w, q):
        pltpu.make_async_copy(msg.at[q], acc_sh.at[rowbuf.at[w]],
                              ssems[q]).wait()

    NSP = CH // 4

    def _chunk(k, carry):
        w0 = s * W + k * CH
        pltpu.sync_copy(col_hbm.at[c, pl.ds(w0, CH)], colbuf)
        pltpu.sync_copy(cols_hbm.at[pl.ds((s * NCHUNK + k) * NSP, NSP)],
                        colbuf_s)
        pltpu.sync_copy(row_hbm.at[pl.ds(w0, CH)], rowbuf)
        _start_gather(0, 0)
        _start_gather(1, 1)
        _wait_gather(0, 0)
        _start_scatter(0, 0)
        _start_gather_s(0, 2)
        _wait_gather(1, 1)
        _start_scatter(1, 1)
        _start_gather(3, 3)

        def _quad(o, carry2):
            wbase = 4 * o + 2
            _wait_gather_s(o, 2)
            _start_scatter(wbase, 2)
            _wait_scatter(wbase - 2, 0)
            _start_gather(wbase + 2, 0)
            _wait_gather(wbase + 1, 3)
            _start_scatter(wbase + 1, 3)
            _wait_scatter(wbase - 1, 1)
            _start_gather(wbase + 3, 1)
            _wait_gather(wbase + 2, 0)
            _start_scatter(wbase + 2, 0)
            _wait_scatter(wbase, 2)
            _start_gather_s(o + 1, 2)
            _wait_gather(wbase + 3, 1)
            _start_scatter(wbase + 3, 1)
            _wait_scatter(wbase + 1, 3)
            _start_gather(wbase + 5, 3)
            return carry2

        lax.fori_loop(0, (CH - 4) // 4, _quad, 0)
        _wait_gather_s(NSP - 1, (CH - 2) % 4)
        _start_scatter(CH - 2, (CH - 2) % 4)
        _wait_scatter(CH - 4, (CH - 4) % 4)
        _wait_gather(CH - 1, (CH - 1) % 4)
        _start_scatter(CH - 1, (CH - 1) % 4)
        _wait_scatter(CH - 3, (CH - 3) % 4)
        for w in range(CH - 2, CH):
            _wait_scatter(w, w % 4)
        return carry

    lax.fori_loop(0, NCHUNK, _chunk, 0)

    plsc.subcore_barrier()

    pltpu.sync_copy(acc_sh.at[pl.ds(r0, ROWS_PER_TILE)],
                    out_hbm.at[pl.ds(r0, ROWS_PER_TILE), pl.ds(c0, DH)])


@jax.jit
def kernel(x, edge_index):
    x2 = jnp.concatenate([x[:, :DH], x[:, DH:]], axis=0)
    col2 = edge_index[1].reshape(E // B, B)
    col3 = jnp.stack([col2, col2 + N])
    cols = col2.reshape(NS, NCHUNK, CH // 4, 4, B)[:, :, :, 2, :]
    cols = cols.reshape(NS * NCHUNK * (CH // 4), B)
    row2 = edge_index[0].reshape(E // B, B)

    mesh = plsc.VectorSubcoreMesh(core_axis_name="c", subcore_axis_name="s")
    out = pl.kernel(
        _body,
        out_type=jax.ShapeDtypeStruct((N, D), jnp.float32),
        mesh=mesh,
        scratch_types=[
            pltpu.VMEM_SHARED((N, DH), jnp.float32),
            pltpu.VMEM_SHARED((N, DH), jnp.float32),
            pltpu.VMEM((CH, B), jnp.int32),
            pltpu.VMEM((CH // 4, B), jnp.int32),
            pltpu.VMEM((CH, B), jnp.int32),
            pltpu.VMEM((NBUF, B, DH), jnp.float32),
            pltpu.SemaphoreType.DMA,
            pltpu.SemaphoreType.DMA,
            pltpu.SemaphoreType.DMA,
            pltpu.SemaphoreType.DMA,
            pltpu.SemaphoreType.DMA,
            pltpu.SemaphoreType.DMA,
            pltpu.SemaphoreType.DMA,
            pltpu.SemaphoreType.DMA,
        ],
        compiler_params=pltpu.CompilerParams(use_tc_tiling_on_sc=False),
    )(x2, col3, cols, row2)
    return out


# speedup vs baseline: 1.2492x; 1.2492x over previous
import jax
import jax.numpy as jnp
from jax import lax
from jax.experimental import pallas as pl
from jax.experimental.pallas import tpu as pltpu
from jax.experimental.pallas import tpu_sc as plsc

N = 10000
E = 320000
D = 128
DH = D // 2
NS = 16
B = 125
W = E // NS // B
NBUF = 6
ROWS_PER_TILE = N // NS
ZROWS = 125
WMAIN = ((W - 3 - 4) // NBUF) * NBUF


def _body(x2_hbm, col_hbm, row_hbm, out_hbm,
          acc_sh, colbuf, rowbuf, msg,
          gsem0, gsem1, gsem2, gsem3, gsem4, gsem5,
          ssem0, ssem1, ssem2, ssem3, ssem4, ssem5):
    c = lax.axis_index("c")
    s = lax.axis_index("s")
    r0 = s * ROWS_PER_TILE
    c0 = c * DH

    zeros16 = jnp.zeros((16,), jnp.float32)

    def _zero_row(r, carry):
        for k in range(DH // 16):
            msg[0, r, pl.ds(k * 16, 16)] = zeros16
        return carry

    lax.fori_loop(0, ZROWS, _zero_row, 0)
    for b in range(ROWS_PER_TILE // ZROWS):
        pltpu.sync_copy(msg.at[0], acc_sh.at[pl.ds(r0 + b * ZROWS, ZROWS)])

    pltpu.sync_copy(col_hbm.at[c, pl.ds(s * W, W)], colbuf)
    pltpu.sync_copy(row_hbm.at[pl.ds(s * W, W)], rowbuf)

    plsc.subcore_barrier()

    gsems = (gsem0, gsem1, gsem2, gsem3, gsem4, gsem5)
    ssems = (ssem0, ssem1, ssem2, ssem3, ssem4, ssem5)

    def _start_gather(w, q):
        pltpu.async_copy(x2_hbm.at[colbuf.at[w]], msg.at[q], gsems[q])

    def _wait_gather(w, q):
        pltpu.make_async_copy(x2_hbm.at[colbuf.at[w]], msg.at[q],
                              gsems[q]).wait()

    def _start_scatter(w, q):
        pltpu.async_copy(msg.at[q], acc_sh.at[rowbuf.at[w]], ssems[q],
                         add=True)

    def _wait_scatter(w, q):
        pltpu.make_async_copy(msg.at[q], acc_sh.at[rowbuf.at[w]],
                              ssems[q]).wait()

    for w in range(3):
        _start_gather(w, w % NBUF)
    for w in range(3):
        _wait_gather(w, w % NBUF)
        _start_scatter(w, w % NBUF)
        _start_gather(w + 3, (w + 3) % NBUF)

    def _hex(o, carry):
        wbase = NBUF * o + 3
        for i in range(NBUF):
            w = wbase + i
            q = (3 + i) % NBUF
            qn = i % NBUF
            _wait_gather(w, q)
            _start_scatter(w, q)
            _wait_scatter(w - 3, qn)
            _start_gather(w + 3, qn)
        return carry

    lax.fori_loop(0, WMAIN // NBUF, _hex, 0)
    for w in range(3 + WMAIN, W - 3):
        _wait_gather(w, w % NBUF)
        _start_scatter(w, w % NBUF)
        _wait_scatter(w - 3, (w - 3) % NBUF)
        _start_gather(w + 3, (w + 3) % NBUF)
    for w in range(W - 3, W):
        _wait_gather(w, w % NBUF)
        _start_scatter(w, w % NBUF)
        _wait_scatter(w - 3, (w - 3) % NBUF)
    for w in range(W - 3, W):
        _wait_scatter(w, w % NBUF)

    plsc.subcore_barrier()

    pltpu.sync_copy(acc_sh.at[pl.ds(r0, ROWS_PER_TILE)],
                    out_hbm.at[pl.ds(r0, ROWS_PER_TILE), pl.ds(c0, DH)])


@jax.jit
def kernel(x, edge_index):
    x2 = jnp.concatenate([x[:, :DH], x[:, DH:]], axis=0)
    col2 = edge_index[1].reshape(E // B, B)
    col3 = jnp.stack([col2, col2 + N])
    row2 = edge_index[0].reshape(E // B, B)

    mesh = plsc.VectorSubcoreMesh(core_axis_name="c", subcore_axis_name="s")
    out = pl.kernel(
        _body,
        out_type=jax.ShapeDtypeStruct((N, D), jnp.float32),
        mesh=mesh,
        scratch_types=[
            pltpu.VMEM_SHARED((N, DH), jnp.float32),
            pltpu.VMEM((W, B), jnp.int32),
            pltpu.VMEM((W, B), jnp.int32),
            pltpu.VMEM((NBUF, B, DH), jnp.float32),
            pltpu.SemaphoreType.DMA,
            pltpu.SemaphoreType.DMA,
            pltpu.SemaphoreType.DMA,
            pltpu.SemaphoreType.DMA,
            pltpu.SemaphoreType.DMA,
            pltpu.SemaphoreType.DMA,
            pltpu.SemaphoreType.DMA,
            pltpu.SemaphoreType.DMA,
            pltpu.SemaphoreType.DMA,
            pltpu.SemaphoreType.DMA,
            pltpu.SemaphoreType.DMA,
            pltpu.SemaphoreType.DMA,
        ],
        compiler_params=pltpu.CompilerParams(use_tc_tiling_on_sc=False),
    )(x2, col3, row2)
    return out


# speedup vs baseline: 1.2811x; 1.0255x over previous
import jax
import jax.numpy as jnp
from jax import lax
from jax.experimental import pallas as pl
from jax.experimental.pallas import tpu as pltpu
from jax.experimental.pallas import tpu_sc as plsc

N = 10000
E = 320000
D = 128
DH = D // 2
NS = 16
B = 125
W = E // NS // B
NBUF = 6
ROWS_PER_TILE = N // NS
ZROWS = 125
WMAIN = ((W - 3 - 4) // NBUF) * NBUF


def _body(x2_hbm, col_hbm, row_hbm, out_hbm,
          acc_sh, colbuf, rowbuf, msg,
          gsem0, gsem1, gsem2, gsem3, gsem4, gsem5,
          ssem0, ssem1, ssem2, ssem3, ssem4, ssem5):
    c = lax.axis_index("c")
    s = lax.axis_index("s")
    r0 = s * ROWS_PER_TILE
    c0 = c * DH

    pltpu.async_copy(col_hbm.at[c, pl.ds(s * W, W)], colbuf, gsem0)
    pltpu.async_copy(row_hbm.at[pl.ds(s * W, W)], rowbuf, gsem1)

    zeros16 = jnp.zeros((16,), jnp.float32)

    def _zero_row(r, carry):
        for k in range(DH // 16):
            msg[0, r, pl.ds(k * 16, 16)] = zeros16
        return carry

    lax.fori_loop(0, ZROWS, _zero_row, 0)
    for b in range(ROWS_PER_TILE // ZROWS):
        pltpu.sync_copy(msg.at[0], acc_sh.at[pl.ds(r0 + b * ZROWS, ZROWS)])

    pltpu.make_async_copy(col_hbm.at[c, pl.ds(s * W, W)], colbuf,
                          gsem0).wait()
    pltpu.make_async_copy(row_hbm.at[pl.ds(s * W, W)], rowbuf,
                          gsem1).wait()

    gsems = (gsem0, gsem1, gsem2, gsem3, gsem4, gsem5)
    ssems = (ssem0, ssem1, ssem2, ssem3, ssem4, ssem5)

    def _start_gather(w, q):
        pltpu.async_copy(x2_hbm.at[colbuf.at[w]], msg.at[q], gsems[q])

    def _wait_gather(w, q):
        pltpu.make_async_copy(x2_hbm.at[colbuf.at[w]], msg.at[q],
                              gsems[q]).wait()

    def _start_scatter(w, q):
        pltpu.async_copy(msg.at[q], acc_sh.at[rowbuf.at[w]], ssems[q],
                         add=True)

    def _wait_scatter(w, q):
        pltpu.make_async_copy(msg.at[q], acc_sh.at[rowbuf.at[w]],
                              ssems[q]).wait()

    for w in range(6):
        _start_gather(w, w % NBUF)

    plsc.subcore_barrier()

    for w in range(3):
        _wait_gather(w, w % NBUF)
        _start_scatter(w, w % NBUF)

    def _hex(o, carry):
        wbase = NBUF * o + 3
        for i in range(NBUF):
            w = wbase + i
            q = (3 + i) % NBUF
            qn = i % NBUF
            _wait_gather(w, q)
            _start_scatter(w, q)
            _wait_scatter(w - 3, qn)
            _start_gather(w + 3, qn)
        return carry

    lax.fori_loop(0, WMAIN // NBUF, _hex, 0)
    for w in range(3 + WMAIN, W - 3):
        _wait_gather(w, w % NBUF)
        _start_scatter(w, w % NBUF)
        _wait_scatter(w - 3, (w - 3) % NBUF)
        _start_gather(w + 3, (w + 3) % NBUF)
    for w in range(W - 3, W):
        _wait_gather(w, w % NBUF)
        _start_scatter(w, w % NBUF)
        _wait_scatter(w - 3, (w - 3) % NBUF)
    for w in range(W - 3, W):
        _wait_scatter(w, w % NBUF)

    plsc.subcore_barrier()

    pltpu.sync_copy(acc_sh.at[pl.ds(r0, ROWS_PER_TILE)],
                    out_hbm.at[pl.ds(r0, ROWS_PER_TILE), pl.ds(c0, DH)])


@jax.jit
def kernel(x, edge_index):
    x2 = jnp.concatenate([x[:, :DH], x[:, DH:]], axis=0)
    col2 = edge_index[1].reshape(E // B, B)
    col3 = jnp.stack([col2, col2 + N])
    row2 = edge_index[0].reshape(E // B, B)

    mesh = plsc.VectorSubcoreMesh(core_axis_name="c", subcore_axis_name="s")
    out = pl.kernel(
        _body,
        out_type=jax.ShapeDtypeStruct((N, D), jnp.float32),
        mesh=mesh,
        scratch_types=[
            pltpu.VMEM_SHARED((N, DH), jnp.float32),
            pltpu.VMEM((W, B), jnp.int32),
            pltpu.VMEM((W, B), jnp.int32),
            pltpu.VMEM((NBUF, B, DH), jnp.float32),
            pltpu.SemaphoreType.DMA,
            pltpu.SemaphoreType.DMA,
            pltpu.SemaphoreType.DMA,
            pltpu.SemaphoreType.DMA,
            pltpu.SemaphoreType.DMA,
            pltpu.SemaphoreType.DMA,
            pltpu.SemaphoreType.DMA,
            pltpu.SemaphoreType.DMA,
            pltpu.SemaphoreType.DMA,
            pltpu.SemaphoreType.DMA,
            pltpu.SemaphoreType.DMA,
            pltpu.SemaphoreType.DMA,
        ],
        compiler_params=pltpu.CompilerParams(use_tc_tiling_on_sc=False),
    )(x2, col3, row2)
    return out
